# baseline (speedup 1.0000x reference)
import functools

import jax
import jax.numpy as jnp
from jax import lax
from jax.experimental import pallas as pl
from jax.experimental.pallas import tpu as pltpu
from jax.experimental.pallas import tpu_sc as plsc

_SMOOTH = 1e-05

_B = 2
_C = 4
_N = 128 * 128 * 128
_NSUB = 16
_VPW = _N // _NSUB
_CH = 8192
_NCHUNK = _VPW // _CH
_L = 16


def _gdice_partials_body(net_hbm, gt_hbm, out_hbm, xb, gb, outb, sem0, sem1):
    cid = lax.axis_index("c")
    sid = lax.axis_index("s")
    vbase = sid * _VPW

    sems = (sem0, sem1)

    def start_chunk(k, buf):
        base = vbase + k * _CH
        copies = []
        for c in range(_C):
            copies.append(
                pltpu.async_copy(
                    net_hbm.at[cid, c, pl.ds(base, _CH)],
                    xb.at[buf, c],
                    sems[buf],
                )
            )
        copies.append(
            pltpu.async_copy(
                gt_hbm.at[cid, pl.ds(base, _CH)], gb.at[buf], sems[buf]
            )
        )
        return copies

    def compute_chunk(buf, accs):
        def body(i, accs):
            (cnt0, cnt1, cnt2, cnt3,
             it0, it1, it2, it3,
             sm0, sm1, sm2, sm3) = accs
            sl = pl.ds(i * _L, _L)
            x0 = xb[buf, 0, sl]
            x1 = xb[buf, 1, sl]
            x2 = xb[buf, 2, sl]
            x3 = xb[buf, 3, sl]
            g = gb[buf, sl]
            m = jnp.maximum(jnp.maximum(x0, x1), jnp.maximum(x2, x3))
            e0 = jnp.exp(x0 - m)
            e1 = jnp.exp(x1 - m)
            e2 = jnp.exp(x2 - m)
            e3 = jnp.exp(x3 - m)
            sinv = 1.0 / ((e0 + e1) + (e2 + e3))
            p0 = e0 * sinv
            p1 = e1 * sinv
            p2 = e2 * sinv
            p3 = e3 * sinv
            m0 = g == 0
            m1 = g == 1
            m2 = g == 2
            m3 = g == 3
            zero = jnp.zeros((_L,), jnp.float32)
            one = jnp.full((_L,), 1.0, jnp.float32)
            return (
                cnt0 + jnp.where(m0, one, zero),
                cnt1 + jnp.where(m1, one, zero),
                cnt2 + jnp.where(m2, one, zero),
                cnt3 + jnp.where(m3, one, zero),
                it0 + jnp.where(m0, p0, zero),
                it1 + jnp.where(m1, p1, zero),
                it2 + jnp.where(m2, p2, zero),
                it3 + jnp.where(m3, p3, zero),
                sm0 + p0,
                sm1 + p1,
                sm2 + p2,
                sm3 + p3,
            )

        return lax.fori_loop(0, _CH // _L, body, accs)

    accs = tuple(jnp.zeros((_L,), jnp.float32) for _ in range(12))

    inflight = start_chunk(0, 0)
    for k in range(_NCHUNK):
        buf = k % 2
        nxt = None
        if k + 1 < _NCHUNK:
            nxt = start_chunk(k + 1, (k + 1) % 2)
        for cp in inflight:
            cp.wait()
        accs = compute_chunk(buf, accs)
        inflight = nxt

    for q in range(12):
        outb[q, :] = accs[q]
    wid = cid * _NSUB + sid
    pltpu.sync_copy(outb, out_hbm.at[wid])


_gdice_partials = functools.partial(
    pl.kernel,
    mesh=plsc.VectorSubcoreMesh(core_axis_name="c", subcore_axis_name="s"),
    out_type=jax.ShapeDtypeStruct((_B * _NSUB, 12, _L), jnp.float32),
    scratch_types=[
        pltpu.VMEM((2, _C, _CH), jnp.float32),
        pltpu.VMEM((2, _CH), jnp.int32),
        pltpu.VMEM((12, _L), jnp.float32),
        pltpu.SemaphoreType.DMA,
        pltpu.SemaphoreType.DMA,
    ],
)(_gdice_partials_body)


@jax.jit
def kernel(net_output, gt):
    net = net_output.reshape(_B, _C, _N)
    labels = gt.reshape(_B, _N).astype(jnp.int32)
    parts = _gdice_partials(net, labels)
    pw = parts.sum(axis=-1)
    pb = pw.reshape(_B, _NSUB, 12).sum(axis=1)
    cnt = pb[:, 0:4]
    inter = pb[:, 4:8]
    smsum = pb[:, 8:12]
    w = 1.0 / (cnt + 1e-10) ** 2
    inter_w = (w * inter).sum(axis=1)
    union_w = (w * (smsum + cnt)).sum(axis=1)
    divided = 1.0 - 2.0 * (inter_w + _SMOOTH) / (union_w + _SMOOTH)
    return divided.mean()


# speedup vs baseline: 1.0142x; 1.0142x over previous
import functools

import jax
import jax.numpy as jnp
from jax import lax
from jax.experimental import pallas as pl
from jax.experimental.pallas import tpu as pltpu
from jax.experimental.pallas import tpu_sc as plsc

_SMOOTH = 1e-05

_B = 2
_C = 4
_N = 128 * 128 * 128
_NSUB = 16
_VPW = _N // _NSUB
_CH = 8192
_NCHUNK = _VPW // _CH
_L = 16
_UNROLL = 4


def _gdice_partials_body(net_hbm, gt_hbm, out_hbm, xb, gb, outb, sem0, sem1):
    cid = lax.axis_index("c")
    sid = lax.axis_index("s")
    vbase = sid * _VPW

    sems = (sem0, sem1)

    def start_chunk(k, buf):
        base = vbase + k * _CH
        copies = []
        for c in range(_C):
            copies.append(
                pltpu.async_copy(
                    net_hbm.at[cid, c, pl.ds(base, _CH)],
                    xb.at[buf, c],
                    sems[buf],
                )
            )
        copies.append(
            pltpu.async_copy(
                gt_hbm.at[cid, pl.ds(base, _CH)], gb.at[buf], sems[buf]
            )
        )
        return copies

    ones = jnp.full((_L,), 1.0, jnp.float32)
    zeros = jnp.zeros((_L,), jnp.float32)

    def compute_chunk(buf, accs):
        def body(i, accs):
            cnt0, cnt1, cnt2, it0, it1, it2, pga, sm0, sm1, sm2 = accs
            for u in range(_UNROLL):
                sl = pl.ds(i * (_L * _UNROLL) + u * _L, _L)
                x0 = xb[buf, 0, sl]
                x1 = xb[buf, 1, sl]
                x2 = xb[buf, 2, sl]
                x3 = xb[buf, 3, sl]
                g = gb[buf, sl]
                m = jnp.maximum(jnp.maximum(x0, x1), jnp.maximum(x2, x3))
                e0 = jnp.exp(x0 - m)
                e1 = jnp.exp(x1 - m)
                e2 = jnp.exp(x2 - m)
                e3 = jnp.exp(x3 - m)
                sinv = 1.0 / ((e0 + e1) + (e2 + e3))
                p0 = e0 * sinv
                p1 = e1 * sinv
                p2 = e2 * sinv
                p3 = e3 * sinv
                m0 = g == 0
                m1 = g == 1
                m2 = g == 2
                pg = jnp.where(m0, p0,
                               jnp.where(m1, p1,
                                         jnp.where(m2, p2, p3)))
                cnt0 = cnt0 + jnp.where(m0, ones, zeros)
                cnt1 = cnt1 + jnp.where(m1, ones, zeros)
                cnt2 = cnt2 + jnp.where(m2, ones, zeros)
                it0 = it0 + jnp.where(m0, p0, zeros)
                it1 = it1 + jnp.where(m1, p1, zeros)
                it2 = it2 + jnp.where(m2, p2, zeros)
                pga = pga + pg
                sm0 = sm0 + p0
                sm1 = sm1 + p1
                sm2 = sm2 + p2
            return (cnt0, cnt1, cnt2, it0, it1, it2, pga, sm0, sm1, sm2)

        return lax.fori_loop(0, _CH // (_L * _UNROLL), body, accs)

    accs = tuple(jnp.zeros((_L,), jnp.float32) for _ in range(10))

    inflight = start_chunk(0, 0)
    for k in range(_NCHUNK):
        buf = k % 2
        nxt = None
        if k + 1 < _NCHUNK:
            nxt = start_chunk(k + 1, (k + 1) % 2)
        for cp in inflight:
            cp.wait()
        accs = compute_chunk(buf, accs)
        inflight = nxt

    for q in range(10):
        outb[q, :] = accs[q]
    wid = cid * _NSUB + sid
    pltpu.sync_copy(outb, out_hbm.at[wid])


_gdice_partials = functools.partial(
    pl.kernel,
    mesh=plsc.VectorSubcoreMesh(core_axis_name="c", subcore_axis_name="s"),
    out_type=jax.ShapeDtypeStruct((_B * _NSUB, 10, _L), jnp.float32),
    scratch_types=[
        pltpu.VMEM((2, _C, _CH), jnp.float32),
        pltpu.VMEM((2, _CH), jnp.int32),
        pltpu.VMEM((10, _L), jnp.float32),
        pltpu.SemaphoreType.DMA,
        pltpu.SemaphoreType.DMA,
    ],
)(_gdice_partials_body)


@jax.jit
def kernel(net_output, gt):
    net = net_output.reshape(_B, _C, _N)
    labels = gt.reshape(_B, _N).astype(jnp.int32)
    parts = _gdice_partials(net, labels)
    pw = parts.sum(axis=-1)
    pb = pw.reshape(_B, _NSUB, 10).sum(axis=1)
    nf = jnp.float32(_N)
    cnt012 = pb[:, 0:3]
    it012 = pb[:, 3:6]
    pga = pb[:, 6:7]
    sm012 = pb[:, 7:10]
    cnt = jnp.concatenate(
        [cnt012, nf - cnt012.sum(axis=1, keepdims=True)], axis=1)
    inter = jnp.concatenate(
        [it012, pga - it012.sum(axis=1, keepdims=True)], axis=1)
    smsum = jnp.concatenate(
        [sm012, nf - sm012.sum(axis=1, keepdims=True)], axis=1)
    w = 1.0 / (cnt + 1e-10) ** 2
    inter_w = (w * inter).sum(axis=1)
    union_w = (w * (smsum + cnt)).sum(axis=1)
    divided = 1.0 - 2.0 * (inter_w + _SMOOTH) / (union_w + _SMOOTH)
    return divided.mean()


# speedup vs baseline: 1.5342x; 1.5127x over previous
import functools

import jax
import jax.numpy as jnp
from jax import lax
from jax.experimental import pallas as pl
from jax.experimental.pallas import tpu as pltpu
from jax.experimental.pallas import tpu_sc as plsc

_SMOOTH = 1e-05

_B = 2
_C = 4
_X = 128
_YH = 64
_Z = 128
_N = _X * _X * _X
_NSUB = 16
_XPW = _X // _NSUB
_NCHUNK = _XPW * 2
_L = 16
_JSL = _Z // _L


def _gdice_partials_body(net_hbm, gt_hbm, out_hbm, xb, gb, outb, sem0, sem1):
    cid = lax.axis_index("c")
    sid = lax.axis_index("s")

    sems = (sem0, sem1)

    def start_chunk(k, buf):
        x = sid * _XPW + k // 2
        y0 = (k % 2) * _YH
        copies = []
        for c in range(_C):
            copies.append(
                pltpu.async_copy(
                    net_hbm.at[cid, c, x, pl.ds(y0, _YH), :],
                    xb.at[buf, c],
                    sems[buf],
                )
            )
        copies.append(
            pltpu.async_copy(
                gt_hbm.at[cid, x, pl.ds(y0, _YH), :], gb.at[buf], sems[buf]
            )
        )
        return copies

    ones = jnp.full((_L,), 1.0, jnp.float32)
    zeros = jnp.zeros((_L,), jnp.float32)

    def compute_chunk(buf, accs):
        def body(r, accs):
            cnt0, cnt1, cnt2, it0, it1, it2, pga, sm0, sm1, sm2 = accs
            for j in range(_JSL):
                sl = pl.ds(j * _L, _L)
                x0 = xb[buf, 0, r, sl]
                x1 = xb[buf, 1, r, sl]
                x2 = xb[buf, 2, r, sl]
                x3 = xb[buf, 3, r, sl]
                g = gb[buf, r, sl]
                m = jnp.maximum(jnp.maximum(x0, x1), jnp.maximum(x2, x3))
                e0 = jnp.exp(x0 - m)
                e1 = jnp.exp(x1 - m)
                e2 = jnp.exp(x2 - m)
                e3 = jnp.exp(x3 - m)
                sinv = 1.0 / ((e0 + e1) + (e2 + e3))
                p0 = e0 * sinv
                p1 = e1 * sinv
                p2 = e2 * sinv
                p3 = e3 * sinv
                m0 = g == 0
                m1 = g == 1
                m2 = g == 2
                pg = jnp.where(m0, p0,
                               jnp.where(m1, p1,
                                         jnp.where(m2, p2, p3)))
                cnt0 = cnt0 + jnp.where(m0, ones, zeros)
                cnt1 = cnt1 + jnp.where(m1, ones, zeros)
                cnt2 = cnt2 + jnp.where(m2, ones, zeros)
                it0 = it0 + jnp.where(m0, p0, zeros)
                it1 = it1 + jnp.where(m1, p1, zeros)
                it2 = it2 + jnp.where(m2, p2, zeros)
                pga = pga + pg
                sm0 = sm0 + p0
                sm1 = sm1 + p1
                sm2 = sm2 + p2
            return (cnt0, cnt1, cnt2, it0, it1, it2, pga, sm0, sm1, sm2)

        return lax.fori_loop(0, _YH, body, accs)

    accs = tuple(jnp.zeros((_L,), jnp.float32) for _ in range(10))

    inflight = start_chunk(0, 0)
    for k in range(_NCHUNK):
        buf = k % 2
        nxt = None
        if k + 1 < _NCHUNK:
            nxt = start_chunk(k + 1, (k + 1) % 2)
        for cp in inflight:
            cp.wait()
        accs = compute_chunk(buf, accs)
        inflight = nxt

    for q in range(10):
        outb[q, :] = accs[q]
    wid = cid * _NSUB + sid
    pltpu.sync_copy(outb, out_hbm.at[wid])


_gdice_partials = functools.partial(
    pl.kernel,
    mesh=plsc.VectorSubcoreMesh(core_axis_name="c", subcore_axis_name="s"),
    out_type=jax.ShapeDtypeStruct((_B * _NSUB, 10, _L), jnp.float32),
    scratch_types=[
        pltpu.VMEM((2, _C, _YH, _Z), jnp.float32),
        pltpu.VMEM((2, _YH, _Z), jnp.int32),
        pltpu.VMEM((10, _L), jnp.float32),
        pltpu.SemaphoreType.DMA,
        pltpu.SemaphoreType.DMA,
    ],
)(_gdice_partials_body)


@jax.jit
def kernel(net_output, gt):
    labels = gt.astype(jnp.int32)
    parts = _gdice_partials(net_output, labels)
    pw = parts.sum(axis=-1)
    pb = pw.reshape(_B, _NSUB, 10).sum(axis=1)
    nf = jnp.float32(_N)
    cnt012 = pb[:, 0:3]
    it012 = pb[:, 3:6]
    pga = pb[:, 6:7]
    sm012 = pb[:, 7:10]
    cnt = jnp.concatenate(
        [cnt012, nf - cnt012.sum(axis=1, keepdims=True)], axis=1)
    inter = jnp.concatenate(
        [it012, pga - it012.sum(axis=1, keepdims=True)], axis=1)
    smsum = jnp.concatenate(
        [sm012, nf - sm012.sum(axis=1, keepdims=True)], axis=1)
    w = 1.0 / (cnt + 1e-10) ** 2
    inter_w = (w * inter).sum(axis=1)
    union_w = (w * (smsum + cnt)).sum(axis=1)
    divided = 1.0 - 2.0 * (inter_w + _SMOOTH) / (union_w + _SMOOTH)
    return divided.mean()


# speedup vs baseline: 2.3827x; 1.5531x over previous
import functools

import jax
import jax.numpy as jnp
from jax import lax
from jax.experimental import pallas as pl
from jax.experimental.pallas import tpu as pltpu
from jax.experimental.pallas import tpu_sc as plsc

_SMOOTH = 1e-05

_B = 2
_C = 4
_X = 128
_YH = 64
_Z = 128
_N = _X * _X * _X
_NSUB = 16
_NW = 2 * _NSUB
_XPW = _X // _NW
_NCHUNK = _XPW * 2
_L = 16
_JSL = _Z // _L
_XB = 8
_TSTEPS = _X // _XB


def _sc_partials_body(net_hbm, gt_hbm, out_hbm, xb, gb, outb, sem0, sem1):
    cid = lax.axis_index("c")
    sid = lax.axis_index("s")
    wid = cid * _NSUB + sid

    sems = (sem0, sem1)

    def start_chunk(k, buf):
        x = wid * _XPW + k // 2
        y0 = (k % 2) * _YH
        copies = []
        for c in range(_C):
            copies.append(
                pltpu.async_copy(
                    net_hbm.at[0, c, x, pl.ds(y0, _YH), :],
                    xb.at[buf, c],
                    sems[buf],
                )
            )
        copies.append(
            pltpu.async_copy(
                gt_hbm.at[0, x, pl.ds(y0, _YH), :], gb.at[buf], sems[buf]
            )
        )
        return copies

    ones = jnp.full((_L,), 1.0, jnp.float32)
    zeros = jnp.zeros((_L,), jnp.float32)

    def compute_chunk(buf, accs):
        def body(r, accs):
            cnt0, cnt1, cnt2, it0, it1, it2, pga, sm0, sm1, sm2 = accs
            for j in range(_JSL):
                sl = pl.ds(j * _L, _L)
                x0 = xb[buf, 0, r, sl]
                x1 = xb[buf, 1, r, sl]
                x2 = xb[buf, 2, r, sl]
                x3 = xb[buf, 3, r, sl]
                g = gb[buf, r, sl]
                m = jnp.maximum(jnp.maximum(x0, x1), jnp.maximum(x2, x3))
                e0 = jnp.exp(x0 - m)
                e1 = jnp.exp(x1 - m)
                e2 = jnp.exp(x2 - m)
                e3 = jnp.exp(x3 - m)
                sinv = 1.0 / ((e0 + e1) + (e2 + e3))
                p0 = e0 * sinv
                p1 = e1 * sinv
                p2 = e2 * sinv
                p3 = e3 * sinv
                m0 = g == 0
                m1 = g == 1
                m2 = g == 2
                pg = jnp.where(m0, p0,
                               jnp.where(m1, p1,
                                         jnp.where(m2, p2, p3)))
                cnt0 = cnt0 + jnp.where(m0, ones, zeros)
                cnt1 = cnt1 + jnp.where(m1, ones, zeros)
                cnt2 = cnt2 + jnp.where(m2, ones, zeros)
                it0 = it0 + jnp.where(m0, p0, zeros)
                it1 = it1 + jnp.where(m1, p1, zeros)
                it2 = it2 + jnp.where(m2, p2, zeros)
                pga = pga + pg
                sm0 = sm0 + p0
                sm1 = sm1 + p1
                sm2 = sm2 + p2
            return (cnt0, cnt1, cnt2, it0, it1, it2, pga, sm0, sm1, sm2)

        return lax.fori_loop(0, _YH, body, accs)

    accs = tuple(jnp.zeros((_L,), jnp.float32) for _ in range(10))

    inflight = start_chunk(0, 0)
    for k in range(_NCHUNK):
        buf = k % 2
        nxt = None
        if k + 1 < _NCHUNK:
            nxt = start_chunk(k + 1, (k + 1) % 2)
        for cp in inflight:
            cp.wait()
        accs = compute_chunk(buf, accs)
        inflight = nxt

    for q in range(10):
        outb[q, :] = accs[q]
    pltpu.sync_copy(outb, out_hbm.at[wid])


_sc_partials = functools.partial(
    pl.kernel,
    mesh=plsc.VectorSubcoreMesh(core_axis_name="c", subcore_axis_name="s"),
    out_type=jax.ShapeDtypeStruct((_NW, 10, _L), jnp.float32),
    scratch_types=[
        pltpu.VMEM((2, _C, _YH, _Z), jnp.float32),
        pltpu.VMEM((2, _YH, _Z), jnp.int32),
        pltpu.VMEM((10, _L), jnp.float32),
        pltpu.SemaphoreType.DMA,
        pltpu.SemaphoreType.DMA,
    ],
)(_sc_partials_body)


def _tc_partials_body(x_ref, g_ref, out_ref, acc_ref):
    i = pl.program_id(0)

    @pl.when(i == 0)
    def _init():
        acc_ref[...] = jnp.zeros_like(acc_ref)

    x = x_ref[0]
    g = g_ref[0]
    m = jnp.max(x, axis=0)
    e = jnp.exp(x - m[None])
    p = e / jnp.sum(e, axis=0)[None]
    for c in range(_C):
        mask = (g == c)
        maskf = mask.astype(jnp.float32)
        acc_ref[c] += jnp.sum(maskf, axis=0)
        acc_ref[4 + c] += jnp.sum(jnp.where(mask, p[c], 0.0), axis=0)
        acc_ref[8 + c] += jnp.sum(p[c], axis=0)

    @pl.when(i == _TSTEPS - 1)
    def _finish():
        out_ref[...] = acc_ref[...]


_tc_partials = pl.pallas_call(
    _tc_partials_body,
    grid=(_TSTEPS,),
    in_specs=[
        pl.BlockSpec((1, _C, _XB, _X, _Z), lambda i: (1, 0, i, 0, 0)),
        pl.BlockSpec((1, _XB, _X, _Z), lambda i: (1, i, 0, 0)),
    ],
    out_specs=pl.BlockSpec((12, _X, _Z), lambda i: (0, 0, 0)),
    out_shape=jax.ShapeDtypeStruct((12, _X, _Z), jnp.float32),
    scratch_shapes=[pltpu.VMEM((12, _X, _Z), jnp.float32)],
)


@jax.jit
def kernel(net_output, gt):
    labels = gt.astype(jnp.int32)
    sc_parts = _sc_partials(net_output, labels)
    tc_parts = _tc_partials(net_output, labels)

    nf = jnp.float32(_N)

    pw = sc_parts.sum(axis=-1).sum(axis=0)
    cnt012, it012, pga, sm012 = pw[0:3], pw[3:6], pw[6], pw[7:10]
    cnt_b0 = jnp.concatenate([cnt012, (nf - cnt012.sum())[None]])
    it_b0 = jnp.concatenate([it012, (pga - it012.sum())[None]])
    sm_b0 = jnp.concatenate([sm012, (nf - sm012.sum())[None]])

    tp = tc_parts.sum(axis=(1, 2))
    cnt_b1, it_b1, sm_b1 = tp[0:4], tp[4:8], tp[8:12]

    cnt = jnp.stack([cnt_b0, cnt_b1])
    inter = jnp.stack([it_b0, it_b1])
    smsum = jnp.stack([sm_b0, sm_b1])
    w = 1.0 / (cnt + 1e-10) ** 2
    inter_w = (w * inter).sum(axis=1)
    union_w = (w * (smsum + cnt)).sum(axis=1)
    divided = 1.0 - 2.0 * (inter_w + _SMOOTH) / (union_w + _SMOOTH)
    return divided.mean()


# speedup vs baseline: 3.4859x; 1.4630x over previous
import functools

import jax
import jax.numpy as jnp
from jax import lax
from jax.experimental import pallas as pl
from jax.experimental.pallas import tpu as pltpu
from jax.experimental.pallas import tpu_sc as plsc

_SMOOTH = 1e-05

_B = 2
_C = 4
_X = 128
_YH = 64
_Z = 128
_N = _X * _X * _X
_NSUB = 16
_NW = 2 * _NSUB
_XS = 64
_XPW = _XS // _NW
_NCHUNK = _XPW * 2
_L = 16
_JSL = _Z // _L
_XB = 8
_S0 = (_X - _XS) // _XB
_TSTEPS = _S0 + _X // _XB


def _sc_partials_body(net_hbm, gt_hbm, out_hbm, xb, gb, outb, sem0, sem1):
    cid = lax.axis_index("c")
    sid = lax.axis_index("s")
    wid = cid * _NSUB + sid

    sems = (sem0, sem1)

    def start_chunk(k, buf):
        x = wid * _XPW + k // 2
        y0 = (k % 2) * _YH
        copies = []
        for c in range(_C):
            copies.append(
                pltpu.async_copy(
                    net_hbm.at[0, c, x, pl.ds(y0, _YH), :],
                    xb.at[buf, c],
                    sems[buf],
                )
            )
        copies.append(
            pltpu.async_copy(
                gt_hbm.at[0, x, pl.ds(y0, _YH), :], gb.at[buf], sems[buf]
            )
        )
        return copies

    ones = jnp.full((_L,), 1.0, jnp.float32)
    zeros = jnp.zeros((_L,), jnp.float32)

    def compute_chunk(buf, accs):
        def body(r, accs):
            cnt0, cnt1, cnt2, it0, it1, it2, pga, sm0, sm1, sm2 = accs
            for j in range(_JSL):
                sl = pl.ds(j * _L, _L)
                x0 = xb[buf, 0, r, sl]
                x1 = xb[buf, 1, r, sl]
                x2 = xb[buf, 2, r, sl]
                x3 = xb[buf, 3, r, sl]
                g = gb[buf, r, sl]
                m = jnp.maximum(jnp.maximum(x0, x1), jnp.maximum(x2, x3))
                e0 = jnp.exp(x0 - m)
                e1 = jnp.exp(x1 - m)
                e2 = jnp.exp(x2 - m)
                e3 = jnp.exp(x3 - m)
                sinv = 1.0 / ((e0 + e1) + (e2 + e3))
                p0 = e0 * sinv
                p1 = e1 * sinv
                p2 = e2 * sinv
                p3 = e3 * sinv
                m0 = g == 0
                m1 = g == 1
                m2 = g == 2
                pg = jnp.where(m0, p0,
                               jnp.where(m1, p1,
                                         jnp.where(m2, p2, p3)))
                cnt0 = cnt0 + jnp.where(m0, ones, zeros)
                cnt1 = cnt1 + jnp.where(m1, ones, zeros)
                cnt2 = cnt2 + jnp.where(m2, ones, zeros)
                it0 = it0 + jnp.where(m0, p0, zeros)
                it1 = it1 + jnp.where(m1, p1, zeros)
                it2 = it2 + jnp.where(m2, p2, zeros)
                pga = pga + pg
                sm0 = sm0 + p0
                sm1 = sm1 + p1
                sm2 = sm2 + p2
            return (cnt0, cnt1, cnt2, it0, it1, it2, pga, sm0, sm1, sm2)

        return lax.fori_loop(0, _YH, body, accs)

    accs = tuple(jnp.zeros((_L,), jnp.float32) for _ in range(10))

    inflight = start_chunk(0, 0)
    for k in range(_NCHUNK):
        buf = k % 2
        nxt = None
        if k + 1 < _NCHUNK:
            nxt = start_chunk(k + 1, (k + 1) % 2)
        for cp in inflight:
            cp.wait()
        accs = compute_chunk(buf, accs)
        inflight = nxt

    for q in range(10):
        outb[q, :] = accs[q]
    pltpu.sync_copy(outb, out_hbm.at[wid])


_sc_partials = functools.partial(
    pl.kernel,
    mesh=plsc.VectorSubcoreMesh(core_axis_name="c", subcore_axis_name="s"),
    out_type=jax.ShapeDtypeStruct((_NW, 10, _L), jnp.float32),
    scratch_types=[
        pltpu.VMEM((2, _C, _YH, _Z), jnp.float32),
        pltpu.VMEM((2, _YH, _Z), jnp.int32),
        pltpu.VMEM((10, _L), jnp.float32),
        pltpu.SemaphoreType.DMA,
        pltpu.SemaphoreType.DMA,
    ],
)(_sc_partials_body)


def _tc_partials_body(x_ref, g_ref, out_ref):
    i = pl.program_id(0)

    @pl.when((i == 0) | (i == _S0))
    def _init():
        out_ref[...] = jnp.zeros_like(out_ref)

    x = x_ref[0]
    g = g_ref[0]
    m = jnp.max(x, axis=0)
    e = jnp.exp(x - m[None])
    p = e / jnp.sum(e, axis=0)[None]
    for c in range(_C):
        mask = (g == c)
        maskf = mask.astype(jnp.float32)
        out_ref[0, c] += jnp.sum(maskf, axis=0)
        out_ref[0, 4 + c] += jnp.sum(jnp.where(mask, p[c], 0.0), axis=0)
        out_ref[0, 8 + c] += jnp.sum(p[c], axis=0)


def _tc_b(i):
    return (i + _X // _XB - _S0) // (_X // _XB)


def _tc_xblk(i):
    b = _tc_b(i)
    return i + _XS // _XB - b * _S0 - b * (_XS // _XB)


_tc_partials = pl.pallas_call(
    _tc_partials_body,
    grid=(_TSTEPS,),
    in_specs=[
        pl.BlockSpec((1, _C, _XB, _X, _Z),
                     lambda i: (_tc_b(i), 0, _tc_xblk(i), 0, 0)),
        pl.BlockSpec((1, _XB, _X, _Z),
                     lambda i: (_tc_b(i), _tc_xblk(i), 0, 0)),
    ],
    out_specs=pl.BlockSpec((1, 12, _X, _Z), lambda i: (_tc_b(i), 0, 0, 0)),
    out_shape=jax.ShapeDtypeStruct((_B, 12, _X, _Z), jnp.float32),
)


@jax.jit
def kernel(net_output, gt):
    labels = gt.astype(jnp.int32)
    sc_parts = _sc_partials(net_output, labels)
    tc_parts = _tc_partials(net_output, labels)

    v_sc = jnp.float32(_XS * _X * _Z)

    pw = sc_parts.sum(axis=-1).sum(axis=0)
    cnt012, it012, pga, sm012 = pw[0:3], pw[3:6], pw[6], pw[7:10]
    cnt_sc = jnp.concatenate([cnt012, (v_sc - cnt012.sum())[None]])
    it_sc = jnp.concatenate([it012, (pga - it012.sum())[None]])
    sm_sc = jnp.concatenate([sm012, (v_sc - sm012.sum())[None]])

    tp = tc_parts.sum(axis=(2, 3))
    cnt_b0 = tp[0, 0:4] + cnt_sc
    it_b0 = tp[0, 4:8] + it_sc
    sm_b0 = tp[0, 8:12] + sm_sc
    cnt_b1, it_b1, sm_b1 = tp[1, 0:4], tp[1, 4:8], tp[1, 8:12]

    cnt = jnp.stack([cnt_b0, cnt_b1])
    inter = jnp.stack([it_b0, it_b1])
    smsum = jnp.stack([sm_b0, sm_b1])
    w = 1.0 / (cnt + 1e-10) ** 2
    inter_w = (w * inter).sum(axis=1)
    union_w = (w * (smsum + cnt)).sum(axis=1)
    divided = 1.0 - 2.0 * (inter_w + _SMOOTH) / (union_w + _SMOOTH)
    return divided.mean()
